# baseline (speedup 1.0000x reference)
import jax
import jax.numpy as jnp
from jax import lax
from jax.experimental import pallas as pl
from jax.experimental.pallas import tpu as pltpu


def _pick_bm(m):
    if m <= 4096:
        return m
    for c in (4096, 3584, 3136, 2048, 1792, 1568, 1024, 896, 784, 512, 448,
              392, 256, 196, 128, 112, 64, 56, 32, 28, 16, 8, 4, 2, 1):
        if m % c == 0:
            return c
    return m


def _pmm(x, w, xid, wid, b=None, res=None, act=None, mean_x=False):
    _, M, K = x.shape
    _, _, N = w.shape
    P = xid.shape[0]
    bm = M if mean_x else _pick_bm(M)
    Mo = 1 if mean_x else M
    bmo = 1 if mean_x else bm
    grid = (P, M // bm)
    has_b = b is not None
    has_r = res is not None

    def body(xid_ref, wid_ref, *refs):
        del xid_ref, wid_ref
        x_ref, w_ref = refs[0], refs[1]
        o_ref = refs[-1]
        xv = x_ref[0]
        if mean_x:
            xv = jnp.mean(xv, axis=0, keepdims=True)
        acc = jnp.dot(xv, w_ref[0], preferred_element_type=jnp.float32)
        i = 2
        if has_b:
            acc = acc + refs[i][0]
            i += 1
        if has_r:
            acc = acc + refs[i][0]
            i += 1
        if act == 'relu':
            acc = jnp.maximum(acc, 0.0)
        elif act == 'sigmoid':
            acc = jax.nn.sigmoid(acc)
        o_ref[0] = acc

    in_specs = [
        pl.BlockSpec((1, bm, K), lambda p, m, xid, wid: (xid[p], m, 0)),
        pl.BlockSpec((1, K, N), lambda p, m, xid, wid: (wid[p], 0, 0)),
    ]
    ops = [x, w]
    if has_b:
        in_specs.append(pl.BlockSpec((1, 1, N), lambda p, m, xid, wid: (wid[p], 0, 0)))
        ops.append(b)
    if has_r:
        in_specs.append(pl.BlockSpec((1, bmo, N), lambda p, m, xid, wid: (p, m, 0)))
        ops.append(res)
    return pl.pallas_call(
        body,
        grid_spec=pltpu.PrefetchScalarGridSpec(
            num_scalar_prefetch=2,
            grid=grid,
            in_specs=in_specs,
            out_specs=pl.BlockSpec((1, bmo, N), lambda p, m, xid, wid: (p, m, 0)),
        ),
        out_shape=jax.ShapeDtypeStruct((P, Mo, N), jnp.float32),
    )(xid.astype(jnp.int32), wid.astype(jnp.int32), *ops)


def _im2col(x, k, stride, pad):
    B, C, H, W = x.shape
    oh = (H + 2 * pad - k) // stride + 1
    ow = (W + 2 * pad - k) // stride + 1
    xp = jnp.pad(x, ((0, 0), (0, 0), (pad, pad), (pad, pad)))
    cols = []
    for kh in range(k):
        for kw in range(k):
            cols.append(xp[:, :, kh:kh + stride * (oh - 1) + 1:stride,
                           kw:kw + stride * (ow - 1) + 1:stride])
    p = jnp.stack(cols, axis=2)
    p = p.reshape(B, C * k * k, oh * ow)
    return p.transpose(0, 2, 1)


def _stack_mm(ws):
    w = jnp.stack(list(ws))
    E, O = w.shape[0], w.shape[1]
    return w.reshape(E, O, -1).transpose(0, 2, 1)


def _fold(w, bn, eps=1e-5):
    s = bn['g'] / jnp.sqrt(bn['v'] + eps)
    return w * s[:, None, None, None], bn['b'] - bn['m'] * s


def _maxpool(x):
    return lax.reduce_window(x, -jnp.inf, lax.max, (1, 3, 3, 1), (1, 2, 2, 1),
                             [(0, 0), (1, 1), (1, 1), (0, 0)])


def _gconv_pairs(x, w, stride, pad, groups):
    P, H, W, C = x.shape
    O = w.shape[1]
    xt = x.transpose(1, 2, 0, 3).reshape(1, H, W, P * C)
    wf = w.reshape(P * O, C // groups, 3, 3)
    y = lax.conv_general_dilated(
        xt, wf, (stride, stride), [(pad, pad), (pad, pad)],
        feature_group_count=P * groups,
        dimension_numbers=('NHWC', 'OIHW', 'NHWC'))
    oh, ow = y.shape[1], y.shape[2]
    return y.reshape(oh, ow, P, O).transpose(2, 0, 1, 3)


def _bilinear_up2(x):
    P, H, W, C = x.shape
    oh, ow = 2 * H, 2 * W
    ys = jnp.arange(oh) * ((H - 1) / (oh - 1))
    xs = jnp.arange(ow) * ((W - 1) / (ow - 1))
    y0 = jnp.floor(ys).astype(jnp.int32)
    y1 = jnp.minimum(y0 + 1, H - 1)
    wy = (ys - y0).astype(x.dtype)
    x0 = jnp.floor(xs).astype(jnp.int32)
    x1 = jnp.minimum(x0 + 1, W - 1)
    wx = (xs - x0).astype(x.dtype)
    r0 = x[:, y0, :, :]
    r1 = x[:, y1, :, :]
    row = r0 * (1.0 - wy)[None, :, None, None] + r1 * wy[None, :, None, None]
    c0 = row[:, :, x0, :]
    c1 = row[:, :, x1, :]
    return c0 * (1.0 - wx)[None, None, :, None] + c1 * wx[None, None, :, None]


def kernel(x, params):
    B = x.shape[0]
    gp = params['gate']
    experts = params['experts']
    E = len(experts)
    P = 2 * B
    arB = jnp.arange(B, dtype=jnp.int32)
    pid = jnp.arange(P, dtype=jnp.int32)
    z1 = jnp.zeros((1,), jnp.int32)

    gpat = _im2col(x, 7, 4, 3)
    wg = gp['c1w'].reshape(gp['c1w'].shape[0], -1).T[None]
    bg = gp['c1b'].reshape(1, 1, -1)
    g = _pmm(gpat, wg, arB, jnp.zeros((B,), jnp.int32), b=bg, act='relu')
    g = g.reshape(B, 56, 56, 32).reshape(B, 4, 14, 4, 14, 32).mean(axis=(2, 4))
    g = g.transpose(0, 3, 1, 2).reshape(B, -1)
    h1 = _pmm(g[None], gp['w1'].T[None], z1, z1,
              b=gp['b1'].reshape(1, 1, -1), act='relu')[0]
    logits = _pmm(h1[None], gp['w2'].T[None], z1, z1,
                  b=gp['b2'].reshape(1, 1, -1))[0]
    topv, topi = lax.top_k(logits, 2)
    gv = jax.nn.softmax(topv, axis=-1)
    eids = topi.reshape(-1).astype(jnp.int32)
    sids = jnp.repeat(arB, 2)

    pat = _im2col(x, 7, 2, 3)
    w0, b0 = zip(*[_fold(e['init_w'], e['init_bn']) for e in experts])
    h = _pmm(pat, _stack_mm(w0), sids, eids,
             b=jnp.stack(b0).reshape(E, 1, -1), act='relu')
    h = _maxpool(h.reshape(P, 112, 112, 64))
    x56 = h.reshape(P, 3136, 64)

    s1 = [e['s1'] for e in experts]
    wa1, ba1 = zip(*[_fold(e['air']['c1w'], e['air']['bn1']) for e in s1])
    a = _pmm(x56, _stack_mm(wa1), pid, eids,
             b=jnp.stack(ba1).reshape(E, 1, -1), act='relu')
    a = _maxpool(a.reshape(P, 56, 56, 8))
    wa2, ba2 = zip(*[_fold(e['air']['c2w'], e['air']['bn2']) for e in s1])
    a = _gconv_pairs(a, jnp.stack(wa2)[eids], 1, 1, 8)
    a = jnp.maximum(a + jnp.stack(ba2)[eids][:, None, None, :], 0.0)
    a = _bilinear_up2(a)
    wa3, ba3 = zip(*[_fold(e['air']['c3w'], e['air']['bn3']) for e in s1])
    att = _pmm(a.reshape(P, 3136, 8), _stack_mm(wa3), pid, eids,
               b=jnp.stack(ba3).reshape(E, 1, -1), act='sigmoid')
    y = _pmm(x56, _stack_mm([e['c1w'] for e in s1]), pid, eids)
    y = _gconv_pairs(y.reshape(P, 56, 56, 16),
                     jnp.stack([e['c2w'] for e in s1])[eids], 1, 1, 16)
    y = y.reshape(P, 3136, 16) * att
    x56 = _pmm(y, _stack_mm([e['c3w'] for e in s1]), pid, eids,
               res=x56, act='relu')

    s2 = [e['s2'] for e in experts]
    y = _pmm(x56, _stack_mm([e['c1w'] for e in s2]), pid, eids)
    y = _gconv_pairs(y.reshape(P, 56, 56, 32),
                     jnp.stack([e['c2w'] for e in s2])[eids], 2, 1, 16)
    idf = x56.reshape(P, 56, 56, 64)[:, ::2, ::2, :].reshape(P, 784, 64)
    idp = _pmm(idf, _stack_mm([e['idw'] for e in s2]), pid, eids)
    x28 = _pmm(y.reshape(P, 784, 32), _stack_mm([e['c3w'] for e in s2]),
               pid, eids, res=idp, act='relu')

    s3 = [e['s3'] for e in experts]
    y = _pmm(x28, _stack_mm([e['c1w'] for e in s3]), pid, eids)
    y = _gconv_pairs(y.reshape(P, 28, 28, 64),
                     jnp.stack([e['c2w'] for e in s3])[eids], 2, 1, 16)
    idf = x28.reshape(P, 28, 28, 128)[:, ::2, ::2, :].reshape(P, 196, 128)
    idp = _pmm(idf, _stack_mm([e['idw'] for e in s3]), pid, eids)
    x14 = _pmm(y.reshape(P, 196, 64), _stack_mm([e['c3w'] for e in s3]),
               pid, eids, res=idp, act='relu')

    wfc = jnp.stack([e['fc_w'] for e in experts]).transpose(0, 2, 1)
    bfc = jnp.stack([e['fc_b'] for e in experts]).reshape(E, 1, -1)
    yfc = _pmm(x14, wfc, pid, eids, b=bfc, mean_x=True)
    nc = yfc.shape[-1]
    out = (yfc.reshape(B, 2, nc) * gv[:, :, None]).sum(axis=1)
    return out.astype(x.dtype)


# speedup vs baseline: 1.4325x; 1.4325x over previous
import numpy as np

import jax
import jax.numpy as jnp
from jax import lax
from jax.experimental import pallas as pl
from jax.experimental.pallas import tpu as pltpu


def _pick_bm(m):
    if m <= 4096:
        return m
    for c in (4096, 3584, 3136, 2048, 1792, 1568, 1024, 896, 784, 512, 448,
              392, 256, 196, 128, 112, 64, 56, 32, 28, 16, 8, 4, 2, 1):
        if m % c == 0:
            return c
    return m


def _pmm(x, w, xid, wid, b=None, res=None, act=None, mean_x=False):
    _, M, K = x.shape
    _, _, N = w.shape
    P = xid.shape[0]
    bm = M if mean_x else _pick_bm(M)
    Mo = 1 if mean_x else M
    bmo = 1 if mean_x else bm
    grid = (P, M // bm)
    has_b = b is not None
    has_r = res is not None

    def body(xid_ref, wid_ref, *refs):
        del xid_ref, wid_ref
        x_ref, w_ref = refs[0], refs[1]
        o_ref = refs[-1]
        xv = x_ref[0]
        if mean_x:
            xv = jnp.mean(xv, axis=0, keepdims=True)
        acc = jnp.dot(xv, w_ref[0], preferred_element_type=jnp.float32)
        i = 2
        if has_b:
            acc = acc + refs[i][0]
            i += 1
        if has_r:
            acc = acc + refs[i][0]
            i += 1
        if act == 'relu':
            acc = jnp.maximum(acc, 0.0)
        elif act == 'sigmoid':
            acc = jax.nn.sigmoid(acc)
        o_ref[0] = acc

    in_specs = [
        pl.BlockSpec((1, bm, K), lambda p, m, xid, wid: (xid[p], m, 0)),
        pl.BlockSpec((1, K, N), lambda p, m, xid, wid: (wid[p], 0, 0)),
    ]
    ops = [x, w]
    if has_b:
        in_specs.append(pl.BlockSpec((1, 1, N), lambda p, m, xid, wid: (wid[p], 0, 0)))
        ops.append(b)
    if has_r:
        in_specs.append(pl.BlockSpec((1, bmo, N), lambda p, m, xid, wid: (p, m, 0)))
        ops.append(res)
    return pl.pallas_call(
        body,
        grid_spec=pltpu.PrefetchScalarGridSpec(
            num_scalar_prefetch=2,
            grid=grid,
            in_specs=in_specs,
            out_specs=pl.BlockSpec((1, bmo, N), lambda p, m, xid, wid: (p, m, 0)),
        ),
        out_shape=jax.ShapeDtypeStruct((P, Mo, N), jnp.float32),
    )(xid.astype(jnp.int32), wid.astype(jnp.int32), *ops)


def _im2col(x, k, stride, pad):
    B, C, H, W = x.shape
    oh = (H + 2 * pad - k) // stride + 1
    ow = (W + 2 * pad - k) // stride + 1
    xp = jnp.pad(x, ((0, 0), (0, 0), (pad, pad), (pad, pad)))
    cols = []
    for kh in range(k):
        for kw in range(k):
            cols.append(xp[:, :, kh:kh + stride * (oh - 1) + 1:stride,
                           kw:kw + stride * (ow - 1) + 1:stride])
    p = jnp.stack(cols, axis=2)
    p = p.reshape(B, C * k * k, oh * ow)
    return p.transpose(0, 2, 1)


def _stack_mm(ws):
    w = jnp.stack(list(ws))
    E, O = w.shape[0], w.shape[1]
    return w.reshape(E, O, -1).transpose(0, 2, 1)


def _fold(w, bn, eps=1e-5):
    s = bn['g'] / jnp.sqrt(bn['v'] + eps)
    return w * s[:, None, None, None], bn['b'] - bn['m'] * s


def _maxpool(x):
    return lax.reduce_window(x, -jnp.inf, lax.max, (1, 3, 3, 1), (1, 2, 2, 1),
                             [(0, 0), (1, 1), (1, 1), (0, 0)])


def _im2col_nhwc(x, k, stride, pad):
    P, H, W, C = x.shape
    oh = (H + 2 * pad - k) // stride + 1
    ow = (W + 2 * pad - k) // stride + 1
    xp = jnp.pad(x, ((0, 0), (pad, pad), (pad, pad), (0, 0)))
    cols = [xp[:, kh:kh + stride * (oh - 1) + 1:stride,
               kw:kw + stride * (ow - 1) + 1:stride, :]
            for kh in range(k) for kw in range(k)]
    p = jnp.stack(cols, axis=-1)
    return p.reshape(P, oh * ow, C * k * k)


def _dense_group_w(ws, groups, cin):
    w = jnp.stack(list(ws))
    E, O, gs = w.shape[0], w.shape[1], w.shape[2]
    og = O // groups
    o = np.arange(O)
    ci = np.arange(gs)
    cidx = (o // og)[:, None] * gs + ci[None, :]
    dense = jnp.zeros((E, O, cin, 3, 3), w.dtype)
    dense = dense.at[:, o[:, None], cidx, :, :].set(w)
    return dense.reshape(E, O, cin * 9).transpose(0, 2, 1)


def _bilinear_up2(x):
    P, H, W, C = x.shape
    oh, ow = 2 * H, 2 * W
    ys = np.arange(oh) * ((H - 1) / (oh - 1))
    xs = np.arange(ow) * ((W - 1) / (ow - 1))
    y0 = np.floor(ys).astype(np.int32)
    y1 = np.minimum(y0 + 1, H - 1)
    wy = jnp.asarray((ys - y0), x.dtype)
    x0 = np.floor(xs).astype(np.int32)
    x1 = np.minimum(x0 + 1, W - 1)
    wx = jnp.asarray((xs - x0), x.dtype)
    r0 = x[:, y0, :, :]
    r1 = x[:, y1, :, :]
    row = r0 * (1.0 - wy)[None, :, None, None] + r1 * wy[None, :, None, None]
    c0 = row[:, :, x0, :]
    c1 = row[:, :, x1, :]
    return c0 * (1.0 - wx)[None, None, :, None] + c1 * wx[None, None, :, None]


def kernel(x, params):
    B = x.shape[0]
    gp = params['gate']
    experts = params['experts']
    E = len(experts)
    P = 2 * B
    arB = jnp.arange(B, dtype=jnp.int32)
    pid = jnp.arange(P, dtype=jnp.int32)
    z1 = jnp.zeros((1,), jnp.int32)

    gpat = _im2col(x, 7, 4, 3)
    wg = gp['c1w'].reshape(gp['c1w'].shape[0], -1).T[None]
    bg = gp['c1b'].reshape(1, 1, -1)
    g = _pmm(gpat, wg, arB, jnp.zeros((B,), jnp.int32), b=bg, act='relu')
    g = g.reshape(B, 56, 56, 32).reshape(B, 4, 14, 4, 14, 32).mean(axis=(2, 4))
    g = g.transpose(0, 3, 1, 2).reshape(B, -1)
    h1 = _pmm(g[None], gp['w1'].T[None], z1, z1,
              b=gp['b1'].reshape(1, 1, -1), act='relu')[0]
    logits = _pmm(h1[None], gp['w2'].T[None], z1, z1,
                  b=gp['b2'].reshape(1, 1, -1))[0]
    topv, topi = lax.top_k(logits, 2)
    gv = jax.nn.softmax(topv, axis=-1)
    eids = topi.reshape(-1).astype(jnp.int32)
    sids = jnp.repeat(arB, 2)

    pat = _im2col(x, 7, 2, 3)
    w0, b0 = zip(*[_fold(e['init_w'], e['init_bn']) for e in experts])
    h = _pmm(pat, _stack_mm(w0), sids, eids,
             b=jnp.stack(b0).reshape(E, 1, -1), act='relu')
    h = _maxpool(h.reshape(P, 112, 112, 64))
    x56 = h.reshape(P, 3136, 64)

    s1 = [e['s1'] for e in experts]
    wa1, ba1 = zip(*[_fold(e['air']['c1w'], e['air']['bn1']) for e in s1])
    a = _pmm(x56, _stack_mm(wa1), pid, eids,
             b=jnp.stack(ba1).reshape(E, 1, -1), act='relu')
    a = _maxpool(a.reshape(P, 56, 56, 8))
    wa2, ba2 = zip(*[_fold(e['air']['c2w'], e['air']['bn2']) for e in s1])
    a = _pmm(_im2col_nhwc(a, 3, 1, 1), _dense_group_w(wa2, 8, 8), pid, eids,
             b=jnp.stack(ba2).reshape(E, 1, -1), act='relu')
    a = _bilinear_up2(a.reshape(P, 28, 28, 8))
    wa3, ba3 = zip(*[_fold(e['air']['c3w'], e['air']['bn3']) for e in s1])
    att = _pmm(a.reshape(P, 3136, 8), _stack_mm(wa3), pid, eids,
               b=jnp.stack(ba3).reshape(E, 1, -1), act='sigmoid')
    y = _pmm(x56, _stack_mm([e['c1w'] for e in s1]), pid, eids)
    y = _pmm(_im2col_nhwc(y.reshape(P, 56, 56, 16), 3, 1, 1),
             _dense_group_w([e['c2w'] for e in s1], 16, 16), pid, eids)
    y = y * att
    x56 = _pmm(y, _stack_mm([e['c3w'] for e in s1]), pid, eids,
               res=x56, act='relu')

    s2 = [e['s2'] for e in experts]
    y = _pmm(x56, _stack_mm([e['c1w'] for e in s2]), pid, eids)
    y = _pmm(_im2col_nhwc(y.reshape(P, 56, 56, 32), 3, 2, 1),
             _dense_group_w([e['c2w'] for e in s2], 16, 32), pid, eids)
    idf = x56.reshape(P, 56, 56, 64)[:, ::2, ::2, :].reshape(P, 784, 64)
    idp = _pmm(idf, _stack_mm([e['idw'] for e in s2]), pid, eids)
    x28 = _pmm(y.reshape(P, 784, 32), _stack_mm([e['c3w'] for e in s2]),
               pid, eids, res=idp, act='relu')

    s3 = [e['s3'] for e in experts]
    y = _pmm(x28, _stack_mm([e['c1w'] for e in s3]), pid, eids)
    y = _pmm(_im2col_nhwc(y.reshape(P, 28, 28, 64), 3, 2, 1),
             _dense_group_w([e['c2w'] for e in s3], 16, 64), pid, eids)
    idf = x28.reshape(P, 28, 28, 128)[:, ::2, ::2, :].reshape(P, 196, 128)
    idp = _pmm(idf, _stack_mm([e['idw'] for e in s3]), pid, eids)
    x14 = _pmm(y.reshape(P, 196, 64), _stack_mm([e['c3w'] for e in s3]),
               pid, eids, res=idp, act='relu')

    wfc = jnp.stack([e['fc_w'] for e in experts]).transpose(0, 2, 1)
    bfc = jnp.stack([e['fc_b'] for e in experts]).reshape(E, 1, -1)
    yfc = _pmm(x14, wfc, pid, eids, b=bfc, mean_x=True)
    nc = yfc.shape[-1]
    out = (yfc.reshape(B, 2, nc) * gv[:, :, None]).sum(axis=1)
    return out.astype(x.dtype)


# speedup vs baseline: 2.7557x; 1.9237x over previous
import numpy as np

import jax
import jax.numpy as jnp
from jax import lax
from jax.experimental import pallas as pl
from jax.experimental.pallas import tpu as pltpu


def _pick_bm(m):
    if m <= 4096:
        return m
    for c in (4096, 3584, 3136, 2048, 1792, 1568, 1024, 896, 784, 512, 448,
              392, 256, 196, 128, 112, 64, 56, 32, 28, 16, 8, 4, 2, 1):
        if m % c == 0:
            return c
    return m


def _pmm(x, w, xid, wid, b=None, res=None, act=None, mean_x=False, x_t=False):
    if x_t:
        _, K, M = x.shape
    else:
        _, M, K = x.shape
    _, _, N = w.shape
    P = xid.shape[0]
    if x_t:
        bm = M
        if M > 4096:
            for c in (2048, 1792, 1536, 1280, 1024, 896, 768, 640, 512, 384,
                      256, 128):
                if M % c == 0:
                    bm = c
                    break
    else:
        bm = M if mean_x else _pick_bm(M)
    Mo = 1 if mean_x else M
    bmo = 1 if mean_x else bm
    grid = (P, M // bm)
    has_b = b is not None
    has_r = res is not None

    def body(xid_ref, wid_ref, *refs):
        del xid_ref, wid_ref
        x_ref, w_ref = refs[0], refs[1]
        o_ref = refs[-1]
        xv = x_ref[0]
        if mean_x:
            xv = jnp.mean(xv, axis=0, keepdims=True)
        if x_t:
            acc = lax.dot_general(xv, w_ref[0], (((0,), (0,)), ((), ())),
                                  preferred_element_type=jnp.float32)
        else:
            acc = jnp.dot(xv, w_ref[0], preferred_element_type=jnp.float32)
        i = 2
        if has_b:
            acc = acc + refs[i][0]
            i += 1
        if has_r:
            acc = acc + refs[i][0]
            i += 1
        if act == 'relu':
            acc = jnp.maximum(acc, 0.0)
        elif act == 'sigmoid':
            acc = jax.nn.sigmoid(acc)
        o_ref[0] = acc

    if x_t:
        x_spec = pl.BlockSpec((1, K, bm), lambda p, m, xid, wid: (xid[p], 0, m))
    else:
        x_spec = pl.BlockSpec((1, bm, K), lambda p, m, xid, wid: (xid[p], m, 0))
    in_specs = [
        x_spec,
        pl.BlockSpec((1, K, N), lambda p, m, xid, wid: (wid[p], 0, 0)),
    ]
    ops = [x, w]
    if has_b:
        in_specs.append(pl.BlockSpec((1, 1, N), lambda p, m, xid, wid: (wid[p], 0, 0)))
        ops.append(b)
    if has_r:
        in_specs.append(pl.BlockSpec((1, bmo, N), lambda p, m, xid, wid: (p, m, 0)))
        ops.append(res)
    return pl.pallas_call(
        body,
        grid_spec=pltpu.PrefetchScalarGridSpec(
            num_scalar_prefetch=2,
            grid=grid,
            in_specs=in_specs,
            out_specs=pl.BlockSpec((1, bmo, N), lambda p, m, xid, wid: (p, m, 0)),
        ),
        out_shape=jax.ShapeDtypeStruct((P, Mo, N), jnp.float32),
    )(xid.astype(jnp.int32), wid.astype(jnp.int32), *ops)


def _im2col_T(x, k, stride, pad):
    B, C, H, W = x.shape
    s = stride
    oh = (H + 2 * pad - k) // s + 1
    hs = H // s
    xr = x.reshape(B, C, hs, s, hs, s).transpose(0, 1, 3, 5, 2, 4)
    pp = k // s + 2
    taps = []
    for kh in range(k):
        dh = kh - pad
        ph, th = dh % s, (dh - dh % s) // s
        for kw in range(k):
            dw = kw - pad
            pw, tw = dw % s, (dw - dw % s) // s
            plane = jnp.pad(xr[:, :, ph, pw],
                            ((0, 0), (0, 0), (pp, pp), (pp, pp)))
            taps.append(plane[:, :, pp + th:pp + th + oh,
                              pp + tw:pp + tw + oh])
    p = jnp.stack(taps, axis=2)
    return p.reshape(B, C * k * k, oh * oh)


def _stack_mm(ws):
    w = jnp.stack(list(ws))
    E, O = w.shape[0], w.shape[1]
    return w.reshape(E, O, -1).transpose(0, 2, 1)


def _fold(w, bn, eps=1e-5):
    s = bn['g'] / jnp.sqrt(bn['v'] + eps)
    return w * s[:, None, None, None], bn['b'] - bn['m'] * s


def _maxpool(x):
    return lax.reduce_window(x, -jnp.inf, lax.max, (1, 3, 3, 1), (1, 2, 2, 1),
                             [(0, 0), (1, 1), (1, 1), (0, 0)])


def _im2col_nhwc(x, k, stride, pad):
    P, H, W, C = x.shape
    oh = (H + 2 * pad - k) // stride + 1
    ow = (W + 2 * pad - k) // stride + 1
    xp = jnp.pad(x, ((0, 0), (pad, pad), (pad, pad), (0, 0)))
    cols = [xp[:, kh:kh + stride * (oh - 1) + 1:stride,
               kw:kw + stride * (ow - 1) + 1:stride, :]
            for kh in range(k) for kw in range(k)]
    p = jnp.stack(cols, axis=-1)
    return p.reshape(P, oh * ow, C * k * k)


def _dense_group_w(ws, groups, cin):
    w = jnp.stack(list(ws))
    E, O, gs = w.shape[0], w.shape[1], w.shape[2]
    og = O // groups
    o = np.arange(O)
    ci = np.arange(gs)
    cidx = (o // og)[:, None] * gs + ci[None, :]
    dense = jnp.zeros((E, O, cin, 3, 3), w.dtype)
    dense = dense.at[:, o[:, None], cidx, :, :].set(w)
    return dense.reshape(E, O, cin * 9).transpose(0, 2, 1)


def _bilinear_up2(x):
    P, H, W, C = x.shape
    oh, ow = 2 * H, 2 * W
    ys = np.arange(oh) * ((H - 1) / (oh - 1))
    xs = np.arange(ow) * ((W - 1) / (ow - 1))
    y0 = np.floor(ys).astype(np.int32)
    y1 = np.minimum(y0 + 1, H - 1)
    wy = jnp.asarray((ys - y0), x.dtype)
    x0 = np.floor(xs).astype(np.int32)
    x1 = np.minimum(x0 + 1, W - 1)
    wx = jnp.asarray((xs - x0), x.dtype)
    r0 = x[:, y0, :, :]
    r1 = x[:, y1, :, :]
    row = r0 * (1.0 - wy)[None, :, None, None] + r1 * wy[None, :, None, None]
    c0 = row[:, :, x0, :]
    c1 = row[:, :, x1, :]
    return c0 * (1.0 - wx)[None, None, :, None] + c1 * wx[None, None, :, None]


def kernel(x, params):
    B = x.shape[0]
    gp = params['gate']
    experts = params['experts']
    E = len(experts)
    P = 2 * B
    arB = jnp.arange(B, dtype=jnp.int32)
    pid = jnp.arange(P, dtype=jnp.int32)
    z1 = jnp.zeros((1,), jnp.int32)

    gpat = _im2col_T(x, 7, 4, 3)
    wg = gp['c1w'].reshape(gp['c1w'].shape[0], -1).T[None]
    bg = gp['c1b'].reshape(1, 1, -1)
    g = _pmm(gpat, wg, arB, jnp.zeros((B,), jnp.int32), b=bg, act='relu',
             x_t=True)
    g = g.reshape(B, 56, 56, 32).reshape(B, 4, 14, 4, 14, 32).mean(axis=(2, 4))
    g = g.transpose(0, 3, 1, 2).reshape(B, -1)
    h1 = _pmm(g[None], gp['w1'].T[None], z1, z1,
              b=gp['b1'].reshape(1, 1, -1), act='relu')[0]
    logits = _pmm(h1[None], gp['w2'].T[None], z1, z1,
                  b=gp['b2'].reshape(1, 1, -1))[0]
    topv, topi = lax.top_k(logits, 2)
    gv = jax.nn.softmax(topv, axis=-1)
    eids = topi.reshape(-1).astype(jnp.int32)
    sids = jnp.repeat(arB, 2)

    pat = _im2col_T(x, 7, 2, 3)
    w0, b0 = zip(*[_fold(e['init_w'], e['init_bn']) for e in experts])
    h = _pmm(pat, _stack_mm(w0), sids, eids,
             b=jnp.stack(b0).reshape(E, 1, -1), act='relu', x_t=True)
    h = _maxpool(h.reshape(P, 112, 112, 64))
    x56 = h.reshape(P, 3136, 64)

    s1 = [e['s1'] for e in experts]
    wa1, ba1 = zip(*[_fold(e['air']['c1w'], e['air']['bn1']) for e in s1])
    a = _pmm(x56, _stack_mm(wa1), pid, eids,
             b=jnp.stack(ba1).reshape(E, 1, -1), act='relu')
    a = _maxpool(a.reshape(P, 56, 56, 8))
    wa2, ba2 = zip(*[_fold(e['air']['c2w'], e['air']['bn2']) for e in s1])
    a = _pmm(_im2col_nhwc(a, 3, 1, 1), _dense_group_w(wa2, 8, 8), pid, eids,
             b=jnp.stack(ba2).reshape(E, 1, -1), act='relu')
    a = _bilinear_up2(a.reshape(P, 28, 28, 8))
    wa3, ba3 = zip(*[_fold(e['air']['c3w'], e['air']['bn3']) for e in s1])
    att = _pmm(a.reshape(P, 3136, 8), _stack_mm(wa3), pid, eids,
               b=jnp.stack(ba3).reshape(E, 1, -1), act='sigmoid')
    y = _pmm(x56, _stack_mm([e['c1w'] for e in s1]), pid, eids)
    y = _pmm(_im2col_nhwc(y.reshape(P, 56, 56, 16), 3, 1, 1),
             _dense_group_w([e['c2w'] for e in s1], 16, 16), pid, eids)
    y = y * att
    x56 = _pmm(y, _stack_mm([e['c3w'] for e in s1]), pid, eids,
               res=x56, act='relu')

    s2 = [e['s2'] for e in experts]
    y = _pmm(x56, _stack_mm([e['c1w'] for e in s2]), pid, eids)
    y = _pmm(_im2col_nhwc(y.reshape(P, 56, 56, 32), 3, 2, 1),
             _dense_group_w([e['c2w'] for e in s2], 16, 32), pid, eids)
    idf = x56.reshape(P, 56, 56, 64)[:, ::2, ::2, :].reshape(P, 784, 64)
    idp = _pmm(idf, _stack_mm([e['idw'] for e in s2]), pid, eids)
    x28 = _pmm(y.reshape(P, 784, 32), _stack_mm([e['c3w'] for e in s2]),
               pid, eids, res=idp, act='relu')

    s3 = [e['s3'] for e in experts]
    y = _pmm(x28, _stack_mm([e['c1w'] for e in s3]), pid, eids)
    y = _pmm(_im2col_nhwc(y.reshape(P, 28, 28, 64), 3, 2, 1),
             _dense_group_w([e['c2w'] for e in s3], 16, 64), pid, eids)
    idf = x28.reshape(P, 28, 28, 128)[:, ::2, ::2, :].reshape(P, 196, 128)
    idp = _pmm(idf, _stack_mm([e['idw'] for e in s3]), pid, eids)
    x14 = _pmm(y.reshape(P, 196, 64), _stack_mm([e['c3w'] for e in s3]),
               pid, eids, res=idp, act='relu')

    wfc = jnp.stack([e['fc_w'] for e in experts]).transpose(0, 2, 1)
    bfc = jnp.stack([e['fc_b'] for e in experts]).reshape(E, 1, -1)
    yfc = _pmm(x14, wfc, pid, eids, b=bfc, mean_x=True)
    nc = yfc.shape[-1]
    out = (yfc.reshape(B, 2, nc) * gv[:, :, None]).sum(axis=1)
    return out.astype(x.dtype)
